# baseline (speedup 1.0000x reference)
import functools

import jax
import jax.numpy as jnp
from jax import lax
from jax.experimental import pallas as pl
from jax.experimental.pallas import tpu as pltpu
from jax.experimental.pallas import tpu_sc as plsc

B, N, D = 64, 32768, 16
NCHUNK = N // 16
NROW = N // 128
NBUF = 4


_GATHER_DNUMS = lax.GatherDimensionNumbers(
    offset_dims=(), collapsed_slice_dims=(0,), start_index_map=(0,))


def _vgather(v, idx16):
    return lax.gather(v, idx16[:, None], _GATHER_DNUMS, (1,),
                      mode=lax.GatherScatterMode.PROMISE_IN_BOUNDS)


def _splat(v, lane):
    return _vgather(v, jnp.full((16,), lane, jnp.int32))


def _serial_scan(v, lanes):
    s = v
    for i in range(1, 16):
        s = jnp.where(lanes == i, s + _splat(s, i - 1), s)
    return s


def _resample_body(w_hbm, off_hbm, state_hbm, out_hbm, rows_hbm, a_v, z_v,
                   off_v, bsum_v, bpref_v, rows_v, slab_v, gsem, osem):
    cid = lax.axis_index("c")
    sid = lax.axis_index("s")
    wid = sid * 2 + cid

    lanes = lax.iota(jnp.int32, 16)
    shift_idx = jnp.maximum(lanes - 1, 0)
    zero16 = jnp.zeros((16,), jnp.int32)
    zero16f = jnp.zeros((16,), jnp.float32)
    lane0 = lanes == 0
    lane15 = lanes == 15

    pltpu.sync_copy(off_hbm, off_v)

    for l in range(2):
        b = wid * 2 + l
        pltpu.sync_copy(w_hbm.at[b], a_v)
        off16 = off_v[pl.ds((b // 16) * 16, 16)]
        u = _splat(off16, b % 16)

        @pl.loop(0, NROW)
        def _e1(r):
            carry = zero16f
            for t in range(8):
                v = jnp.exp(a_v[pl.ds(r * 128 + t * 16, 16)])
                v = v + jnp.where(lane0, carry, zero16f)
                s = _serial_scan(v, lanes)
                a_v[pl.ds(r * 128 + t * 16, 16)] = s
                carry = _splat(s, 15)
            plsc.store_scatter(bsum_v, [jnp.full((16,), r, jnp.int32)], carry,
                               mask=lane0)

        @pl.loop(0, 2, init_carry=zero16f)
        def _e2(q, sup):
            carry = zero16f
            for t in range(8):
                v = bsum_v[pl.ds(q * 128 + t * 16, 16)]
                v = v + jnp.where(lane0, carry, zero16f)
                s = _serial_scan(v, lanes)
                bsum_v[pl.ds(q * 128 + t * 16, 16)] = s + sup
                carry = _splat(s, 15)
            return _splat(s + sup, 15)

        @pl.loop(0, 16, init_carry=zero16f)
        def _e2s(k, prevvec):
            v = bsum_v[pl.ds(k * 16, 16)]
            sh = _vgather(v, shift_idx)
            ex = jnp.where(lane0, _splat(prevvec, 15), sh)
            ex = jnp.where((k == 0) & lane0, zero16f, ex)
            bpref_v[pl.ds(k * 16, 16)] = ex
            return v

        def _in_start(s):
            for dt in range(2):
                pltpu.async_copy(state_hbm.at[b, dt, s],
                                 slab_v.at[s % NBUF, pl.ds(dt * 8, 8)],
                                 gsem.at[s % NBUF])

        def _in_wait(s):
            for dt in range(2):
                pltpu.make_async_copy(state_hbm.at[b, dt, s],
                                      slab_v.at[s % NBUF, pl.ds(dt * 8, 8)],
                                      gsem.at[s % NBUF]).wait()

        def _rows_start(s):
            pltpu.async_copy(rows_v.at[s % NBUF],
                             rows_hbm.at[b, pl.ds(s * 128, 128)],
                             osem.at[s % NBUF])

        def _rows_wait(s):
            pltpu.make_async_copy(rows_v.at[s % NBUF],
                                  rows_hbm.at[b, pl.ds(s * 128, 128)],
                                  osem.at[s % NBUF]).wait()

        for s in range(NBUF):
            _in_start(s)

        @pl.loop(0, NROW)
        def _pa(s):
            _in_wait(s)

            @pl.when(s >= NBUF)
            def _():
                _rows_wait(s - NBUF)

            buf = jnp.full((16,), s % NBUF, jnp.int32)
            for g in range(8):
                ivec = jnp.full((16,), g * 16, jnp.int32) + lanes
                for d in range(16):
                    v = slab_v[s % NBUF, d, pl.ds(g * 16, 16)]
                    plsc.store_scatter(
                        rows_v, [buf, ivec, jnp.full((16,), d, jnp.int32)], v)
            _rows_start(s)

            nxt = s + NBUF

            @pl.when(nxt < NROW)
            def _():
                _in_start(nxt)

        @pl.loop(NROW - NBUF, NROW)
        def _draina(s):
            _rows_wait(s)

        @pl.loop(0, NCHUNK)
        def _p0(k):
            z_v[pl.ds(k * 16, 16)] = zero16

        @pl.loop(0, NCHUNK, init_carry=zero16)
        def _p1(k, fvec_prev):
            c = k // 8
            bp16 = bpref_v[pl.ds((c // 16) * 16, 16)]
            cum16 = a_v[pl.ds(k * 16, 16)] + _splat(bp16, c % 16)
            cum16 = jnp.minimum(cum16, 1.0)
            a16 = cum16 * float(N)
            a16 = jnp.where((k == NCHUNK - 1) & lane15, float(N), a16)
            t = a16 - u
            i0 = jnp.clip(t.astype(jnp.int32), -1, N - 1)
            for _ in range(2):
                pf = (i0 + 1).astype(jnp.float32) + u
                up = (pf <= a16) & (i0 < N - 1)
                i0 = jnp.where(up, i0 + 1, i0)
            for _ in range(2):
                pf = i0.astype(jnp.float32) + u
                dn = (pf > a16) & (i0 >= 0)
                i0 = jnp.where(dn, i0 - 1, i0)
            f = i0 + 1
            fshift = _vgather(f, shift_idx)
            fprev = jnp.where(lanes == 0, _splat(fvec_prev, 15), fshift)
            mask = f > fprev
            jval = jnp.full((16,), k * 16, jnp.int32) + lanes
            plsc.store_scatter(z_v, [fprev], jval, mask=mask)
            return f

        @pl.loop(0, NROW, init_carry=zero16)
        def _p2(r, cvec):
            lcs = []
            for t in range(8):
                lcs.append(plsc.cummax(z_v[pl.ds(r * 128 + t * 16, 16)]))
            cur = cvec
            for t in range(8):
                out = jnp.maximum(lcs[t], cur)
                z_v[pl.ds(r * 128 + t * 16, 16)] = out
                cur = _splat(out, 15)
            return cur

        def _gather_start(s):
            pltpu.async_copy(rows_hbm.at[b].at[z_v.at[pl.ds(s * 128, 128)]],
                             rows_v.at[s % NBUF], gsem.at[s % NBUF])

        def _gather_wait(s):
            pltpu.make_async_copy(rows_hbm.at[b].at[z_v.at[pl.ds(s * 128, 128)]],
                                  rows_v.at[s % NBUF], gsem.at[s % NBUF]).wait()

        def _out_start(s):
            for dt in range(2):
                pltpu.async_copy(slab_v.at[s % NBUF, pl.ds(dt * 8, 8)],
                                 out_hbm.at[b, dt, s], osem.at[s % NBUF])

        def _out_wait(s):
            for dt in range(2):
                pltpu.make_async_copy(slab_v.at[s % NBUF, pl.ds(dt * 8, 8)],
                                      out_hbm.at[b, dt, s],
                                      osem.at[s % NBUF]).wait()

        def _transpose(s):
            buf = jnp.full((16,), s % NBUF, jnp.int32)
            for d in range(16):
                dvec = jnp.full((16,), d, jnp.int32)
                for g in range(8):
                    ivec = jnp.full((16,), g * 16, jnp.int32) + lanes
                    got = plsc.load_gather(rows_v, [buf, ivec, dvec])
                    slab_v[s % NBUF, d, pl.ds(g * 16, 16)] = got

        for s in range(NBUF):
            _gather_start(s)

        @pl.loop(0, NROW)
        def _p3(s):
            _gather_wait(s)

            @pl.when(s >= NBUF)
            def _():
                _out_wait(s - NBUF)

            _transpose(s)
            _out_start(s)

            nxt = s + NBUF

            @pl.when(nxt < NROW)
            def _():
                _gather_start(nxt)

        @pl.loop(NROW - NBUF, NROW)
        def _drain(s):
            _out_wait(s)


def _sc_resample(weight, offset, state5):
    mesh = plsc.VectorSubcoreMesh(core_axis_name="c", subcore_axis_name="s")
    f = pl.kernel(
        _resample_body,
        out_type=jax.ShapeDtypeStruct((B, 2, NROW, 8, 128), jnp.float32),
        mesh=mesh,
        compiler_params=pltpu.CompilerParams(
            needs_layout_passes=False, use_tc_tiling_on_sc=False),
        scratch_types=[
            pltpu.HBM((B, N, D), jnp.float32),
            pltpu.VMEM((N,), jnp.float32),
            pltpu.VMEM((N,), jnp.int32),
            pltpu.VMEM((B,), jnp.float32),
            pltpu.VMEM((NROW,), jnp.float32),
            pltpu.VMEM((NROW,), jnp.float32),
            pltpu.VMEM((NBUF, 128, D), jnp.float32),
            pltpu.VMEM((NBUF, D, 128), jnp.float32),
            pltpu.SemaphoreType.DMA((NBUF,)),
            pltpu.SemaphoreType.DMA((NBUF,)),
        ],
    )
    return f(weight, offset, state5)


def kernel(state, weight, offset):
    n = weight.shape[1]
    state5 = (state.transpose(0, 2, 1)
              .reshape(B, 2, 8, NROW, 128)
              .transpose(0, 1, 3, 2, 4))
    out5 = _sc_resample(weight, offset, state5)
    out_state = (out5.transpose(0, 1, 3, 2, 4)
                 .reshape(B, D, N)
                 .transpose(0, 2, 1))
    out_weight = jnp.full(weight.shape, -jnp.log(float(n)), weight.dtype)
    return out_state, out_weight


# speedup vs baseline: 1.2353x; 1.2353x over previous
import functools

import jax
import jax.numpy as jnp
from jax import lax
from jax.experimental import pallas as pl
from jax.experimental.pallas import tpu as pltpu
from jax.experimental.pallas import tpu_sc as plsc

B, N, D = 64, 32768, 16
NCHUNK = N // 16
NROW = N // 128
NBUF = 4


_GATHER_DNUMS = lax.GatherDimensionNumbers(
    offset_dims=(), collapsed_slice_dims=(0,), start_index_map=(0,))


def _vgather(v, idx16):
    return lax.gather(v, idx16[:, None], _GATHER_DNUMS, (1,),
                      mode=lax.GatherScatterMode.PROMISE_IN_BOUNDS)


def _splat(v, lane):
    return _vgather(v, jnp.full((16,), lane, jnp.int32))


def _serial_scan(v, lanes):
    s = v
    for i in range(1, 16):
        s = jnp.where(lanes == i, s + _splat(s, i - 1), s)
    return s


def _resample_body(w_hbm, off_hbm, state_hbm, out_hbm, rows_hbm, a_v, z_v,
                   off_v, bsum_v, bpref_v, rows_v, slab_v, gsem, osem):
    cid = lax.axis_index("c")
    sid = lax.axis_index("s")
    wid = sid * 2 + cid

    lanes = lax.iota(jnp.int32, 16)
    shift_idx = jnp.maximum(lanes - 1, 0)
    zero16 = jnp.zeros((16,), jnp.int32)
    zero16f = jnp.zeros((16,), jnp.float32)
    lane0 = lanes == 0
    lane15 = lanes == 15

    pltpu.sync_copy(off_hbm, off_v)

    for l in range(2):
        b = wid * 2 + l
        pltpu.sync_copy(w_hbm.at[b], a_v)
        off16 = off_v[pl.ds((b // 16) * 16, 16)]
        u = _splat(off16, b % 16)

        @pl.loop(0, NROW // 2)
        def _e1(r):
            carries = [zero16f, zero16f]
            for t in range(8):
                for hh in range(2):
                    base = (r * 2 + hh) * 128 + t * 16
                    v = jnp.exp(a_v[pl.ds(base, 16)])
                    v = v + jnp.where(lane0, carries[hh], zero16f)
                    sv = _serial_scan(v, lanes)
                    a_v[pl.ds(base, 16)] = sv
                    carries[hh] = _splat(sv, 15)
            for hh in range(2):
                plsc.store_scatter(bsum_v,
                                   [jnp.full((16,), r * 2 + hh, jnp.int32)],
                                   carries[hh], mask=lane0)

        @pl.loop(0, 2, init_carry=zero16f)
        def _e2(q, sup):
            carry = zero16f
            for t in range(8):
                v = bsum_v[pl.ds(q * 128 + t * 16, 16)]
                v = v + jnp.where(lane0, carry, zero16f)
                s = _serial_scan(v, lanes)
                bsum_v[pl.ds(q * 128 + t * 16, 16)] = s + sup
                carry = _splat(s, 15)
            return _splat(s + sup, 15)

        @pl.loop(0, 16, init_carry=zero16f)
        def _e2s(k, prevvec):
            v = bsum_v[pl.ds(k * 16, 16)]
            sh = _vgather(v, shift_idx)
            ex = jnp.where(lane0, _splat(prevvec, 15), sh)
            ex = jnp.where((k == 0) & lane0, zero16f, ex)
            bpref_v[pl.ds(k * 16, 16)] = ex
            return v

        def _in_start(s):
            for dt in range(2):
                pltpu.async_copy(state_hbm.at[b, dt, s],
                                 slab_v.at[pl.ds((s % NBUF) * 16 + dt * 8, 8)],
                                 gsem.at[s % NBUF])

        def _in_wait(s):
            for dt in range(2):
                pltpu.make_async_copy(state_hbm.at[b, dt, s],
                                      slab_v.at[pl.ds((s % NBUF) * 16 + dt * 8, 8)],
                                      gsem.at[s % NBUF]).wait()

        def _rows_start(s):
            pltpu.async_copy(rows_v.at[pl.ds((s % NBUF) * 128, 128)],
                             rows_hbm.at[b, pl.ds(s * 128, 128)],
                             osem.at[s % NBUF])

        def _rows_wait(s):
            pltpu.make_async_copy(rows_v.at[pl.ds((s % NBUF) * 128, 128)],
                                  rows_hbm.at[b, pl.ds(s * 128, 128)],
                                  osem.at[s % NBUF]).wait()

        for s in range(NBUF):
            _in_start(s)

        @pl.loop(0, NROW)
        def _pa(s):
            _in_wait(s)

            @pl.when(s >= NBUF)
            def _():
                _rows_wait(s - NBUF)

            for g in range(8):
                rowvec = jnp.full((16,), (s % NBUF) * 128 + g * 16,
                                  jnp.int32) + lanes
                for d in range(16):
                    v = slab_v[(s % NBUF) * 16 + d, pl.ds(g * 16, 16)]
                    plsc.store_scatter(
                        rows_v, [rowvec, jnp.full((16,), d, jnp.int32)], v)
            _rows_start(s)

            nxt = s + NBUF

            @pl.when(nxt < NROW)
            def _():
                _in_start(nxt)

        @pl.loop(NROW - NBUF, NROW)
        def _draina(s):
            _rows_wait(s)

        @pl.loop(0, NCHUNK)
        def _p0(k):
            z_v[pl.ds(k * 16, 16)] = zero16

        @pl.loop(0, NCHUNK, init_carry=zero16)
        def _p1(k, fvec_prev):
            c = k // 8
            bp16 = bpref_v[pl.ds((c // 16) * 16, 16)]
            cum16 = a_v[pl.ds(k * 16, 16)] + _splat(bp16, c % 16)
            cum16 = jnp.minimum(cum16, 1.0)
            a16 = cum16 * float(N)
            a16 = jnp.where((k == NCHUNK - 1) & lane15, float(N), a16)
            t = a16 - u
            i0 = jnp.clip(t.astype(jnp.int32), -1, N - 1)
            for _ in range(2):
                pf = (i0 + 1).astype(jnp.float32) + u
                up = (pf <= a16) & (i0 < N - 1)
                i0 = jnp.where(up, i0 + 1, i0)
            for _ in range(2):
                pf = i0.astype(jnp.float32) + u
                dn = (pf > a16) & (i0 >= 0)
                i0 = jnp.where(dn, i0 - 1, i0)
            f = i0 + 1
            fshift = _vgather(f, shift_idx)
            fprev = jnp.where(lanes == 0, _splat(fvec_prev, 15), fshift)
            mask = f > fprev
            jval = jnp.full((16,), k * 16, jnp.int32) + lanes
            plsc.store_scatter(z_v, [fprev], jval, mask=mask)
            return f

        @pl.loop(0, NROW, init_carry=zero16)
        def _p2(r, cvec):
            lcs = []
            for t in range(8):
                lcs.append(plsc.cummax(z_v[pl.ds(r * 128 + t * 16, 16)]))
            cur = cvec
            for t in range(8):
                out = jnp.maximum(lcs[t], cur)
                z_v[pl.ds(r * 128 + t * 16, 16)] = out
                cur = _splat(out, 15)
            return cur

        def _gather_start(s):
            pltpu.async_copy(rows_hbm.at[b].at[z_v.at[pl.ds(s * 128, 128)]],
                             rows_v.at[pl.ds((s % NBUF) * 128, 128)],
                             gsem.at[s % NBUF])

        def _gather_wait(s):
            pltpu.make_async_copy(rows_hbm.at[b].at[z_v.at[pl.ds(s * 128, 128)]],
                                  rows_v.at[pl.ds((s % NBUF) * 128, 128)],
                                  gsem.at[s % NBUF]).wait()

        def _out_start(s):
            for dt in range(2):
                pltpu.async_copy(slab_v.at[pl.ds((s % NBUF) * 16 + dt * 8, 8)],
                                 out_hbm.at[b, dt, s], osem.at[s % NBUF])

        def _out_wait(s):
            for dt in range(2):
                pltpu.make_async_copy(slab_v.at[pl.ds((s % NBUF) * 16 + dt * 8, 8)],
                                      out_hbm.at[b, dt, s],
                                      osem.at[s % NBUF]).wait()

        def _transpose(s):
            for g in range(8):
                rowvec = jnp.full((16,), (s % NBUF) * 128 + g * 16,
                                  jnp.int32) + lanes
                for d in range(16):
                    got = plsc.load_gather(
                        rows_v, [rowvec, jnp.full((16,), d, jnp.int32)])
                    slab_v[(s % NBUF) * 16 + d, pl.ds(g * 16, 16)] = got

        for s in range(NBUF):
            _gather_start(s)

        @pl.loop(0, NROW)
        def _p3(s):
            _gather_wait(s)

            @pl.when(s >= NBUF)
            def _():
                _out_wait(s - NBUF)

            _transpose(s)
            _out_start(s)

            nxt = s + NBUF

            @pl.when(nxt < NROW)
            def _():
                _gather_start(nxt)

        @pl.loop(NROW - NBUF, NROW)
        def _drain(s):
            _out_wait(s)


def _sc_resample(weight, offset, state5):
    mesh = plsc.VectorSubcoreMesh(core_axis_name="c", subcore_axis_name="s")
    f = pl.kernel(
        _resample_body,
        out_type=jax.ShapeDtypeStruct((B, 2, NROW, 8, 128), jnp.float32),
        mesh=mesh,
        compiler_params=pltpu.CompilerParams(
            needs_layout_passes=False, use_tc_tiling_on_sc=False),
        scratch_types=[
            pltpu.HBM((B, N, D), jnp.float32),
            pltpu.VMEM((N,), jnp.float32),
            pltpu.VMEM((N,), jnp.int32),
            pltpu.VMEM((B,), jnp.float32),
            pltpu.VMEM((NROW,), jnp.float32),
            pltpu.VMEM((NROW,), jnp.float32),
            pltpu.VMEM((NBUF * 128, D), jnp.float32),
            pltpu.VMEM((NBUF * 16, 128), jnp.float32),
            pltpu.SemaphoreType.DMA((NBUF,)),
            pltpu.SemaphoreType.DMA((NBUF,)),
        ],
    )
    return f(weight, offset, state5)


def kernel(state, weight, offset):
    n = weight.shape[1]
    state5 = (state.transpose(0, 2, 1)
              .reshape(B, 2, 8, NROW, 128)
              .transpose(0, 1, 3, 2, 4))
    out5 = _sc_resample(weight, offset, state5)
    out_state = (out5.transpose(0, 1, 3, 2, 4)
                 .reshape(B, D, N)
                 .transpose(0, 2, 1))
    out_weight = jnp.full(weight.shape, -jnp.log(float(n)), weight.dtype)
    return out_state, out_weight


# speedup vs baseline: 1.2541x; 1.0152x over previous
import functools

import jax
import jax.numpy as jnp
from jax import lax
from jax.experimental import pallas as pl
from jax.experimental.pallas import tpu as pltpu
from jax.experimental.pallas import tpu_sc as plsc

B, N, D = 64, 32768, 16
NCHUNK = N // 16
NROW = N // 128
NBUF = 6


_GATHER_DNUMS = lax.GatherDimensionNumbers(
    offset_dims=(), collapsed_slice_dims=(0,), start_index_map=(0,))


def _vgather(v, idx16):
    return lax.gather(v, idx16[:, None], _GATHER_DNUMS, (1,),
                      mode=lax.GatherScatterMode.PROMISE_IN_BOUNDS)


def _splat(v, lane):
    return _vgather(v, jnp.full((16,), lane, jnp.int32))


def _serial_scan(v, lanes):
    s = v
    for i in range(1, 16):
        s = jnp.where(lanes == i, s + _splat(s, i - 1), s)
    return s


def _resample_body(w_hbm, off_hbm, state_hbm, out_hbm, rows_hbm, a_v, z_v,
                   off_v, bsum_v, bpref_v, rows_v, slab_v, gsem, osem):
    cid = lax.axis_index("c")
    sid = lax.axis_index("s")
    wid = sid * 2 + cid

    lanes = lax.iota(jnp.int32, 16)
    shift_idx = jnp.maximum(lanes - 1, 0)
    zero16 = jnp.zeros((16,), jnp.int32)
    zero16f = jnp.zeros((16,), jnp.float32)
    lane0 = lanes == 0
    lane15 = lanes == 15

    pltpu.sync_copy(off_hbm, off_v)

    for l in range(2):
        b = wid * 2 + l
        pltpu.sync_copy(w_hbm.at[b], a_v)
        off16 = off_v[pl.ds((b // 16) * 16, 16)]
        u = _splat(off16, b % 16)

        @pl.loop(0, NROW // 2)
        def _e1(r):
            carries = [zero16f, zero16f]
            for t in range(8):
                for hh in range(2):
                    base = (r * 2 + hh) * 128 + t * 16
                    v = jnp.exp(a_v[pl.ds(base, 16)])
                    v = v + jnp.where(lane0, carries[hh], zero16f)
                    sv = _serial_scan(v, lanes)
                    a_v[pl.ds(base, 16)] = sv
                    carries[hh] = _splat(sv, 15)
            for hh in range(2):
                plsc.store_scatter(bsum_v,
                                   [jnp.full((16,), r * 2 + hh, jnp.int32)],
                                   carries[hh], mask=lane0)

        @pl.loop(0, 2, init_carry=zero16f)
        def _e2(q, sup):
            carry = zero16f
            for t in range(8):
                v = bsum_v[pl.ds(q * 128 + t * 16, 16)]
                v = v + jnp.where(lane0, carry, zero16f)
                s = _serial_scan(v, lanes)
                bsum_v[pl.ds(q * 128 + t * 16, 16)] = s + sup
                carry = _splat(s, 15)
            return _splat(s + sup, 15)

        @pl.loop(0, 16, init_carry=zero16f)
        def _e2s(k, prevvec):
            v = bsum_v[pl.ds(k * 16, 16)]
            sh = _vgather(v, shift_idx)
            ex = jnp.where(lane0, _splat(prevvec, 15), sh)
            ex = jnp.where((k == 0) & lane0, zero16f, ex)
            bpref_v[pl.ds(k * 16, 16)] = ex
            return v

        def _in_start(s):
            for dt in range(2):
                pltpu.async_copy(state_hbm.at[b, dt, s],
                                 slab_v.at[pl.ds((s % NBUF) * 16 + dt * 8, 8)],
                                 gsem.at[s % NBUF])

        def _in_wait(s):
            for dt in range(2):
                pltpu.make_async_copy(state_hbm.at[b, dt, s],
                                      slab_v.at[pl.ds((s % NBUF) * 16 + dt * 8, 8)],
                                      gsem.at[s % NBUF]).wait()

        def _rows_start(s):
            pltpu.async_copy(rows_v.at[pl.ds((s % NBUF) * 128, 128)],
                             rows_hbm.at[b, pl.ds(s * 128, 128)],
                             osem.at[s % NBUF])

        def _rows_wait(s):
            pltpu.make_async_copy(rows_v.at[pl.ds((s % NBUF) * 128, 128)],
                                  rows_hbm.at[b, pl.ds(s * 128, 128)],
                                  osem.at[s % NBUF]).wait()

        for s in range(NBUF):
            _in_start(s)

        @pl.loop(0, NROW)
        def _pa(s):
            _in_wait(s)

            @pl.when(s >= NBUF)
            def _():
                _rows_wait(s - NBUF)

            for g in range(8):
                rowvec = jnp.full((16,), (s % NBUF) * 128 + g * 16,
                                  jnp.int32) + lanes
                for d in range(16):
                    v = slab_v[(s % NBUF) * 16 + d, pl.ds(g * 16, 16)]
                    plsc.store_scatter(
                        rows_v, [rowvec, jnp.full((16,), d, jnp.int32)], v)
            _rows_start(s)

            nxt = s + NBUF

            @pl.when(nxt < NROW)
            def _():
                _in_start(nxt)

        @pl.loop(NROW - NBUF, NROW)
        def _draina(s):
            _rows_wait(s)

        @pl.loop(0, NCHUNK, unroll=8)
        def _p0(k):
            z_v[pl.ds(k * 16, 16)] = zero16

        @pl.loop(0, NCHUNK, init_carry=zero16, unroll=2)
        def _p1(k, fvec_prev):
            c = k // 8
            bp16 = bpref_v[pl.ds((c // 16) * 16, 16)]
            cum16 = a_v[pl.ds(k * 16, 16)] + _splat(bp16, c % 16)
            cum16 = jnp.minimum(cum16, 1.0)
            a16 = cum16 * float(N)
            a16 = jnp.where((k == NCHUNK - 1) & lane15, float(N), a16)
            t = a16 - u
            i0 = jnp.clip(t.astype(jnp.int32), -1, N - 1)
            for _ in range(2):
                pf = (i0 + 1).astype(jnp.float32) + u
                up = (pf <= a16) & (i0 < N - 1)
                i0 = jnp.where(up, i0 + 1, i0)
            for _ in range(2):
                pf = i0.astype(jnp.float32) + u
                dn = (pf > a16) & (i0 >= 0)
                i0 = jnp.where(dn, i0 - 1, i0)
            f = i0 + 1
            fshift = _vgather(f, shift_idx)
            fprev = jnp.where(lanes == 0, _splat(fvec_prev, 15), fshift)
            mask = f > fprev
            jval = jnp.full((16,), k * 16, jnp.int32) + lanes
            plsc.store_scatter(z_v, [fprev], jval, mask=mask)
            return f

        @pl.loop(0, NROW, init_carry=zero16)
        def _p2(r, cvec):
            lcs = []
            for t in range(8):
                lcs.append(plsc.cummax(z_v[pl.ds(r * 128 + t * 16, 16)]))
            cur = cvec
            for t in range(8):
                out = jnp.maximum(lcs[t], cur)
                z_v[pl.ds(r * 128 + t * 16, 16)] = out
                cur = _splat(out, 15)
            return cur

        def _gather_start(s):
            pltpu.async_copy(rows_hbm.at[b].at[z_v.at[pl.ds(s * 128, 128)]],
                             rows_v.at[pl.ds((s % NBUF) * 128, 128)],
                             gsem.at[s % NBUF])

        def _gather_wait(s):
            pltpu.make_async_copy(rows_hbm.at[b].at[z_v.at[pl.ds(s * 128, 128)]],
                                  rows_v.at[pl.ds((s % NBUF) * 128, 128)],
                                  gsem.at[s % NBUF]).wait()

        def _out_start(s):
            for dt in range(2):
                pltpu.async_copy(slab_v.at[pl.ds((s % NBUF) * 16 + dt * 8, 8)],
                                 out_hbm.at[b, dt, s], osem.at[s % NBUF])

        def _out_wait(s):
            for dt in range(2):
                pltpu.make_async_copy(slab_v.at[pl.ds((s % NBUF) * 16 + dt * 8, 8)],
                                      out_hbm.at[b, dt, s],
                                      osem.at[s % NBUF]).wait()

        def _transpose(s):
            for g in range(8):
                rowvec = jnp.full((16,), (s % NBUF) * 128 + g * 16,
                                  jnp.int32) + lanes
                for d in range(16):
                    got = plsc.load_gather(
                        rows_v, [rowvec, jnp.full((16,), d, jnp.int32)])
                    slab_v[(s % NBUF) * 16 + d, pl.ds(g * 16, 16)] = got

        for s in range(NBUF):
            _gather_start(s)

        @pl.loop(0, NROW)
        def _p3(s):
            _gather_wait(s)

            @pl.when(s >= NBUF)
            def _():
                _out_wait(s - NBUF)

            _transpose(s)
            _out_start(s)

            nxt = s + NBUF

            @pl.when(nxt < NROW)
            def _():
                _gather_start(nxt)

        @pl.loop(NROW - NBUF, NROW)
        def _drain(s):
            _out_wait(s)


def _sc_resample(weight, offset, state5):
    mesh = plsc.VectorSubcoreMesh(core_axis_name="c", subcore_axis_name="s")
    f = pl.kernel(
        _resample_body,
        out_type=jax.ShapeDtypeStruct((B, 2, NROW, 8, 128), jnp.float32),
        mesh=mesh,
        compiler_params=pltpu.CompilerParams(
            needs_layout_passes=False, use_tc_tiling_on_sc=False),
        scratch_types=[
            pltpu.HBM((B, N, D), jnp.float32),
            pltpu.VMEM((N,), jnp.float32),
            pltpu.VMEM((N,), jnp.int32),
            pltpu.VMEM((B,), jnp.float32),
            pltpu.VMEM((NROW,), jnp.float32),
            pltpu.VMEM((NROW,), jnp.float32),
            pltpu.VMEM((NBUF * 128, D), jnp.float32),
            pltpu.VMEM((NBUF * 16, 128), jnp.float32),
            pltpu.SemaphoreType.DMA((NBUF,)),
            pltpu.SemaphoreType.DMA((NBUF,)),
        ],
    )
    return f(weight, offset, state5)


def kernel(state, weight, offset):
    n = weight.shape[1]
    state5 = (state.transpose(0, 2, 1)
              .reshape(B, 2, 8, NROW, 128)
              .transpose(0, 1, 3, 2, 4))
    out5 = _sc_resample(weight, offset, state5)
    out_state = (out5.transpose(0, 1, 3, 2, 4)
                 .reshape(B, D, N)
                 .transpose(0, 2, 1))
    out_weight = jnp.full(weight.shape, -jnp.log(float(n)), weight.dtype)
    return out_state, out_weight
